# Initial kernel scaffold; baseline (speedup 1.0000x reference)
import jax
import jax.numpy as jnp
from jax.experimental import pallas as pl


def kernel(x, edge_index, W1, a1_src, a1_dst, b1, W2, a2_src, a2_dst, b2):
    raise NotImplementedError("write your pallas kernel here")


# speedup vs baseline: 4.9517x; 4.9517x over previous
import jax
import jax.numpy as jnp
from jax import lax
from jax.experimental import pallas as pl
from jax.experimental.pallas import tpu as pltpu
from jax.experimental.pallas import tpu_sc as plsc

N = 10000
E = 160000
HEADS = 4
DIN = 256
D1 = 1024
DOUT = 256
PADH = 16
DENS = 8
CH = 128
PADW = 128
EB = 128
NBLK = E // EB
NC = 2
NS = 16
NW = NC * NS
NP = 10240
RPT = NP // NS
HNP = NP // 2
HRT = HNP // NS
ACC = HNP + 8
RB = 400
GRID = N // RB
_F32 = jnp.float32
_I32 = jnp.int32


def _elu(x):
    return jnp.where(x > 0, x, jnp.exp(x) - 1.0)


def _tc_a_body(x_ref, w1_ref, a1s_ref, a1d_ref, *outs):
    h = jnp.dot(x_ref[...], w1_ref[...], preferred_element_type=_F32)
    for c in range(8):
        outs[c][...] = h[:, c * CH:(c + 1) * CH]
    outs[8][...] = jnp.dot(h, a1s_ref[...], preferred_element_type=_F32)
    outs[9][...] = jnp.dot(h, a1d_ref[...], preferred_element_type=_F32)


def _tc_c_body(m0, m1, m2, m3, m4, m5, m6, m7, dp_ref, b1_ref, w2_ref,
               a2s_ref, a2d_ref, r1_ref, h2c0, h2c1, as2, ad2):
    msg = jnp.concatenate([r[...] for r in (m0, m1, m2, m3, m4, m5, m6, m7)],
                          axis=1)
    d = jnp.sum(dp_ref[...], axis=0)
    r = 1.0 / (d + 1e-16)
    rep = jnp.dot(r, r1_ref[...], preferred_element_type=_F32)
    t = _elu(msg * rep + b1_ref[...])
    h2 = jnp.dot(t, w2_ref[...], preferred_element_type=_F32)
    h2c0[...] = h2[:, :CH]
    h2c1[...] = h2[:, CH:]
    as2[...] = jnp.dot(h2, a2s_ref[...], preferred_element_type=_F32)
    ad2[...] = jnp.dot(h2, a2d_ref[...], preferred_element_type=_F32)


def _tc_e_body(m0, m1, dp_ref, b2_ref, r2_ref, out_ref):
    msg = jnp.concatenate([m0[...], m1[...]], axis=1)
    d = jnp.sum(dp_ref[...], axis=0)
    r = 1.0 / (d + 1e-16)
    rep = jnp.dot(r, r2_ref[...], preferred_element_type=_F32)
    out_ref[...] = _elu(msg * rep + b2_ref[...])


def _rep_spec(shape):
    return pl.BlockSpec(shape, lambda i: tuple(0 for _ in shape))


def _tc_a(x, W1, A1s, A1d):
    return pl.pallas_call(
        _tc_a_body,
        grid=(GRID,),
        in_specs=[
            pl.BlockSpec((RB, DIN), lambda i: (i, 0)),
            _rep_spec((DIN, D1)),
            _rep_spec((D1, PADW)),
            _rep_spec((D1, PADW)),
        ],
        out_specs=[pl.BlockSpec((RB, CH), lambda i: (i, 0))] * 8
        + [pl.BlockSpec((RB, PADW), lambda i: (i, 0))] * 2,
        out_shape=[jax.ShapeDtypeStruct((N, CH), _F32)] * 8
        + [jax.ShapeDtypeStruct((N, PADW), _F32)] * 2,
    )(x, W1, A1s, A1d)


def _tc_c(m1c, dp1, b1r, W2, A2s, A2d, R1):
    return pl.pallas_call(
        _tc_c_body,
        grid=(GRID,),
        in_specs=[pl.BlockSpec((RB, CH), lambda i: (i, 0))] * 8 + [
            pl.BlockSpec((NW, RB, DENS), lambda i: (0, i, 0)),
            _rep_spec((1, D1)),
            _rep_spec((D1, DOUT)),
            _rep_spec((DOUT, PADW)),
            _rep_spec((DOUT, PADW)),
            _rep_spec((DENS, D1)),
        ],
        out_specs=[pl.BlockSpec((RB, CH), lambda i: (i, 0))] * 2
        + [pl.BlockSpec((RB, PADW), lambda i: (i, 0))] * 2,
        out_shape=[jax.ShapeDtypeStruct((N, CH), _F32)] * 2
        + [jax.ShapeDtypeStruct((N, PADW), _F32)] * 2,
    )(*m1c, dp1, b1r, W2, A2s, A2d, R1)


def _tc_e(m2c, dp2, b2r, R2):
    return pl.pallas_call(
        _tc_e_body,
        grid=(GRID,),
        in_specs=[pl.BlockSpec((RB, CH), lambda i: (i, 0))] * 2 + [
            pl.BlockSpec((NW, RB, DENS), lambda i: (0, i, 0)),
            _rep_spec((1, DOUT)),
            _rep_spec((DENS, DOUT)),
        ],
        out_specs=pl.BlockSpec((RB, DOUT), lambda i: (i, 0)),
        out_shape=jax.ShapeDtypeStruct((N, DOUT), _F32),
    )(*m2c, dp2, b2r, R2)


_MESH = plsc.VectorSubcoreMesh(core_axis_name="c", subcore_axis_name="s")
_B1_ITERS = -(-NBLK // NW)
_B2_ITERS = -(-NBLK // NS)


def _sc_b1_body(asrc, adst, srci, dsti, z8, w_hbm, dpart,
                si, di, ar, dr, ws, den_v):
    cid = lax.axis_index("c")
    sid = lax.axis_index("s")
    wid = sid * NC + cid
    pltpu.sync_copy(z8, den_v)

    def blkbody(i, carry):
        blk = i * NW + wid

        @pl.when(blk < NBLK)
        def _():
            base = blk * EB
            pltpu.sync_copy(srci.at[pl.ds(base, EB)], si)
            pltpu.sync_copy(dsti.at[pl.ds(base, EB)], di)
            pltpu.sync_copy(asrc.at[si], ar)
            pltpu.sync_copy(adst.at[di], dr)
            for g in range(EB // 16):
                rows = lax.iota(_I32, 16) + (g * 16)
                di16 = di[pl.ds(g * 16, 16)]
                for h in range(HEADS):
                    colh = jnp.full((16,), h, _I32)
                    av = plsc.load_gather(ar, [rows, colh])
                    dv = plsc.load_gather(dr, [rows, colh])
                    v = av + dv
                    v = jnp.where(v > 0, v, 0.2 * v)
                    w = jnp.exp(v)
                    plsc.addupdate_scatter(den_v, [di16 * DENS + h], w)
                    plsc.store_scatter(ws, [rows * PADH + h], w)
            pltpu.sync_copy(ws, w_hbm.at[pl.ds(base * PADH, EB * PADH)])
        return carry

    lax.fori_loop(0, _B1_ITERS, blkbody, 0)
    pltpu.sync_copy(den_v, dpart.at[wid])


def _sc_b1(asrc, adst, srci, dsti, z8):
    return pl.kernel(
        _sc_b1_body,
        out_type=[
            jax.ShapeDtypeStruct((E * PADH,), _F32),
            jax.ShapeDtypeStruct((NW, NP * DENS), _F32),
        ],
        mesh=_MESH,
        compiler_params=pltpu.CompilerParams(needs_layout_passes=False),
        scratch_types=[
            pltpu.VMEM((EB,), _I32),
            pltpu.VMEM((EB,), _I32),
            pltpu.VMEM((EB, PADW), _F32),
            pltpu.VMEM((EB, PADW), _F32),
            pltpu.VMEM((EB * PADH,), _F32),
            pltpu.VMEM((NP * DENS,), _F32),
        ],
    )(asrc, adst, srci, dsti, z8)


def _make_sc_b2(nchunk, chunk_heads, chunks_per_core):
    def body(srci, dsti, w_hbm, zc, *rest):
        hc = rest[:nchunk]
        mout = rest[nchunk:2 * nchunk]
        si, di, dl, wr, hr, zv, acc_sp = rest[2 * nchunk:]
        cid = lax.axis_index("c")
        sid = lax.axis_index("s")
        pltpu.sync_copy(zc.at[pl.ds(0, HRT)], zv)

        for c in range(nchunk):

            @pl.when(c // chunks_per_core == cid)
            def _(c=c):
                lanes = lax.iota(_I32, PADH)
                hmask = jnp.where(lanes == chunk_heads[c], 1.0, 0.0)
                for half in range(2):
                    lo = half * HNP
                    pltpu.sync_copy(zv, acc_sp.at[pl.ds(sid * HRT, HRT)])

                    @pl.when(sid == 0)
                    def _():
                        pltpu.sync_copy(zv.at[pl.ds(0, 8)],
                                        acc_sp.at[pl.ds(HNP, 8)])

                    plsc.subcore_barrier()

                    def blkbody(i, carry):
                        blk = i * NS + sid

                        @pl.when(blk < NBLK)
                        def _():
                            base = blk * EB
                            pltpu.sync_copy(srci.at[pl.ds(base, EB)], si)
                            pltpu.sync_copy(dsti.at[pl.ds(base, EB)], di)
                            pltpu.sync_copy(
                                w_hbm.at[pl.ds(base * PADH, EB * PADH)], wr)
                            pltpu.sync_copy(hc[c].at[si], hr)

                            def ebody(j, c2):
                                s = jnp.sum(wr[pl.ds(j * PADH, PADH)] * hmask)
                                for f in range(CH // 16):
                                    sl = pl.ds(f * 16, 16)
                                    hr[j, sl] = hr[j, sl] * s
                                return c2

                            lax.fori_loop(0, EB, ebody, 0)
                            for k in range(EB // 16):
                                sl = pl.ds(k * 16, 16)
                                t = di[sl] - lo
                                ok = (t >= 0) & (t < HNP)
                                dl[sl] = jnp.where(ok, t, HNP)
                            pltpu.sync_copy(hr, acc_sp.at[dl], add=True)
                        return carry

                    lax.fori_loop(0, _B2_ITERS, blkbody, 0)
                    plsc.subcore_barrier()
                    pltpu.sync_copy(
                        acc_sp.at[pl.ds(sid * HRT, HRT)],
                        mout[c].at[pl.ds(lo + sid * HRT, HRT)])
                    plsc.subcore_barrier()

    def run(srci, dsti, w_hbm, zc, hchunks):
        return pl.kernel(
            body,
            out_type=[jax.ShapeDtypeStruct((NP, CH), _F32)] * nchunk,
            mesh=_MESH,
            compiler_params=pltpu.CompilerParams(needs_layout_passes=False),
            scratch_types=[
                pltpu.VMEM((EB,), _I32),
                pltpu.VMEM((EB,), _I32),
                pltpu.VMEM((EB,), _I32),
                pltpu.VMEM((EB * PADH,), _F32),
                pltpu.VMEM((EB, CH), _F32),
                pltpu.VMEM((HRT, CH), _F32),
                pltpu.VMEM_SHARED((ACC, CH), _F32),
            ],
        )(srci, dsti, w_hbm, zc, *hchunks)

    return run


_sc_b2_l1 = _make_sc_b2(8, (0, 0, 1, 1, 2, 2, 3, 3), 4)
_sc_b2_l2 = _make_sc_b2(2, (0, 0), 1)


@jax.jit
def kernel(x, edge_index, W1, a1_src, a1_dst, b1, W2, a2_src, a2_dst, b2):
    src = edge_index[0].astype(_I32)
    dst = edge_index[1].astype(_I32)

    eye = jnp.eye(HEADS, dtype=_F32)
    A1s = jnp.pad((eye[:, None, :] * a1_src.astype(_F32)[:, :, None])
                  .reshape(D1, HEADS), ((0, 0), (0, PADW - HEADS)))
    A1d = jnp.pad((eye[:, None, :] * a1_dst.astype(_F32)[:, :, None])
                  .reshape(D1, HEADS), ((0, 0), (0, PADW - HEADS)))
    A2s = jnp.pad(a2_src.astype(_F32).T, ((0, 0), (0, PADW - 1)))
    A2d = jnp.pad(a2_dst.astype(_F32).T, ((0, 0), (0, PADW - 1)))
    R1 = jnp.pad(jnp.repeat(eye, D1 // HEADS, axis=1),
                 ((0, DENS - HEADS), (0, 0)))
    R2 = jnp.pad(jnp.ones((1, DOUT), _F32), ((0, DENS - 1), (0, 0)))
    b1r = b1.astype(_F32).reshape(1, D1)
    b2r = b2.astype(_F32).reshape(1, DOUT)
    z8 = jnp.zeros((NP * DENS,), _F32)
    zc = jnp.zeros((RPT, CH), _F32)

    outs = _tc_a(x.astype(_F32), W1.astype(_F32), A1s, A1d)
    h1c, as1, ad1 = outs[:8], outs[8], outs[9]

    w1e, dp1 = _sc_b1(as1, ad1, src, dst, z8)
    dp1 = dp1.reshape(NW, NP, DENS)
    m1c = _sc_b2_l1(src, dst, w1e, zc, h1c)

    h2c0, h2c1, as2, ad2 = _tc_c(m1c, dp1, b1r, W2.astype(_F32),
                                 A2s, A2d, R1)

    w2e, dp2 = _sc_b1(as2, ad2, src, dst, z8)
    dp2 = dp2.reshape(NW, NP, DENS)
    m2c = _sc_b2_l2(src, dst, w2e, zc, (h2c0, h2c1))

    return _tc_e(m2c, dp2, b2r, R2)


# speedup vs baseline: 5.9223x; 1.1960x over previous
import jax
import jax.numpy as jnp
from jax import lax
from jax.experimental import pallas as pl
from jax.experimental.pallas import tpu as pltpu
from jax.experimental.pallas import tpu_sc as plsc

N = 10000
E = 160000
HEADS = 4
DIN = 256
D1 = 1024
DOUT = 256
PADH = 16
DENS = 8
CH = 128
PADW = 128
EB = 128
NBLK = E // EB
EPAD = 163840
NBLKP = EPAD // EB
NC = 2
NS = 16
NW = NC * NS
NP = 10240
RPT = NP // NS
HNP = NP // 2
HRT = HNP // NS
ACC = HNP + 8
RB = 400
GRID = N // RB
_F32 = jnp.float32
_I32 = jnp.int32


def _elu(x):
    return jnp.where(x > 0, x, jnp.exp(x) - 1.0)


def _tc_a_body(x_ref, w1_ref, a1s_ref, a1d_ref, *outs):
    h = jnp.dot(x_ref[...], w1_ref[...], preferred_element_type=_F32)
    for c in range(8):
        outs[c][...] = h[:, c * CH:(c + 1) * CH]
    outs[8][...] = jnp.dot(h, a1s_ref[...], preferred_element_type=_F32)
    outs[9][...] = jnp.dot(h, a1d_ref[...], preferred_element_type=_F32)


def _tc_c_body(m0, m1, m2, m3, m4, m5, m6, m7, dp_ref, b1_ref, w2_ref,
               a2s_ref, a2d_ref, r1_ref, h2c0, h2c1, as2, ad2):
    msg = jnp.concatenate([r[...] for r in (m0, m1, m2, m3, m4, m5, m6, m7)],
                          axis=1)
    d = jnp.sum(dp_ref[...], axis=0)
    r = 1.0 / (d + 1e-16)
    rep = jnp.dot(r, r1_ref[...], preferred_element_type=_F32)
    t = _elu(msg * rep + b1_ref[...])
    h2 = jnp.dot(t, w2_ref[...], preferred_element_type=_F32)
    h2c0[...] = h2[:, :CH]
    h2c1[...] = h2[:, CH:]
    as2[...] = jnp.dot(h2, a2s_ref[...], preferred_element_type=_F32)
    ad2[...] = jnp.dot(h2, a2d_ref[...], preferred_element_type=_F32)


def _tc_e_body(m0, m1, dp_ref, b2_ref, r2_ref, out_ref):
    msg = jnp.concatenate([m0[...], m1[...]], axis=1)
    d = jnp.sum(dp_ref[...], axis=0)
    r = 1.0 / (d + 1e-16)
    rep = jnp.dot(r, r2_ref[...], preferred_element_type=_F32)
    out_ref[...] = _elu(msg * rep + b2_ref[...])


def _rep_spec(shape):
    return pl.BlockSpec(shape, lambda i: tuple(0 for _ in shape))


def _tc_a(x, W1, A1s, A1d):
    return pl.pallas_call(
        _tc_a_body,
        grid=(GRID,),
        in_specs=[
            pl.BlockSpec((RB, DIN), lambda i: (i, 0)),
            _rep_spec((DIN, D1)),
            _rep_spec((D1, PADW)),
            _rep_spec((D1, PADW)),
        ],
        out_specs=[pl.BlockSpec((RB, CH), lambda i: (i, 0))] * 8
        + [pl.BlockSpec((RB, PADW), lambda i: (i, 0))] * 2,
        out_shape=[jax.ShapeDtypeStruct((N, CH), _F32)] * 8
        + [jax.ShapeDtypeStruct((N, PADW), _F32)] * 2,
    )(x, W1, A1s, A1d)


def _tc_c(m1c, dp1, b1r, W2, A2s, A2d, R1):
    return pl.pallas_call(
        _tc_c_body,
        grid=(GRID,),
        in_specs=[pl.BlockSpec((RB, CH), lambda i: (i, 0))] * 8 + [
            pl.BlockSpec((NW, RB, DENS), lambda i: (0, i, 0)),
            _rep_spec((1, D1)),
            _rep_spec((D1, DOUT)),
            _rep_spec((DOUT, PADW)),
            _rep_spec((DOUT, PADW)),
            _rep_spec((DENS, D1)),
        ],
        out_specs=[pl.BlockSpec((RB, CH), lambda i: (i, 0))] * 2
        + [pl.BlockSpec((RB, PADW), lambda i: (i, 0))] * 2,
        out_shape=[jax.ShapeDtypeStruct((N, CH), _F32)] * 2
        + [jax.ShapeDtypeStruct((N, PADW), _F32)] * 2,
    )(*m1c, dp1, b1r, W2, A2s, A2d, R1)


def _tc_e(m2c, dp2, b2r, R2):
    return pl.pallas_call(
        _tc_e_body,
        grid=(GRID,),
        in_specs=[pl.BlockSpec((RB, CH), lambda i: (i, 0))] * 2 + [
            pl.BlockSpec((NW, RB, DENS), lambda i: (0, i, 0)),
            _rep_spec((1, DOUT)),
            _rep_spec((DENS, DOUT)),
        ],
        out_specs=pl.BlockSpec((RB, DOUT), lambda i: (i, 0)),
        out_shape=jax.ShapeDtypeStruct((N, DOUT), _F32),
    )(*m2c, dp2, b2r, R2)


_MESH = plsc.VectorSubcoreMesh(core_axis_name="c", subcore_axis_name="s")
_B1_ITERS = NBLKP // NW
_B2_ITERS = NBLKP // NS


def _sc_b1_body(asrc, adst, srci, dsti, z8, w_hbm, dpart,
                si, di, ar, dr, ws, den_v):
    cid = lax.axis_index("c")
    sid = lax.axis_index("s")
    wid = sid * NC + cid
    pltpu.sync_copy(z8, den_v)

    def blkbody(i, carry):
        base = (i * NW + wid) * EB
        pltpu.sync_copy(srci.at[pl.ds(base, EB)], si)
        pltpu.sync_copy(dsti.at[pl.ds(base, EB)], di)
        pltpu.sync_copy(asrc.at[si], ar)
        pltpu.sync_copy(adst.at[di], dr)
        for g in range(EB // 16):
            rows = lax.iota(_I32, 16) + (g * 16)
            di16 = di[pl.ds(g * 16, 16)]
            for h in range(HEADS):
                colh = jnp.full((16,), h, _I32)
                av = plsc.load_gather(ar, [rows, colh])
                dv = plsc.load_gather(dr, [rows, colh])
                v = av + dv
                v = jnp.where(v > 0, v, 0.2 * v)
                w = jnp.exp(v)
                plsc.addupdate_scatter(den_v, [di16 * DENS + h], w)
                plsc.store_scatter(ws, [rows * PADH + h], w)
        pltpu.sync_copy(ws, w_hbm.at[pl.ds(base * PADH, EB * PADH)])
        return carry

    lax.fori_loop(0, _B1_ITERS, blkbody, 0)
    pltpu.sync_copy(den_v, dpart.at[wid])


def _sc_b1(asrc, adst, srci, dsti, z8):
    return pl.kernel(
        _sc_b1_body,
        out_type=[
            jax.ShapeDtypeStruct((EPAD * PADH,), _F32),
            jax.ShapeDtypeStruct((NW, NP * DENS), _F32),
        ],
        mesh=_MESH,
        compiler_params=pltpu.CompilerParams(needs_layout_passes=False),
        scratch_types=[
            pltpu.VMEM((EB,), _I32),
            pltpu.VMEM((EB,), _I32),
            pltpu.VMEM((EB, PADW), _F32),
            pltpu.VMEM((EB, PADW), _F32),
            pltpu.VMEM((EB * PADH,), _F32),
            pltpu.VMEM((NP * DENS,), _F32),
        ],
    )(asrc, adst, srci, dsti, z8)


def _make_sc_b2(nchunk, chunk_heads, chunks_per_core):
    def body(srci, dsti, w_hbm, zc, *rest):
        hc = rest[:nchunk]
        mout = rest[nchunk:2 * nchunk]
        (si0, si1, di0, di1, dl, wr0, wr1, hr0, hr1, zv,
         smi0, smi1, smg0, smg1, acc_sp) = rest[2 * nchunk:]
        si = (si0, si1)
        di = (di0, di1)
        wr = (wr0, wr1)
        hr = (hr0, hr1)
        smi = (smi0, smi1)
        smg = (smg0, smg1)
        cid = lax.axis_index("c")
        sid = lax.axis_index("s")
        pltpu.sync_copy(zc.at[pl.ds(0, HRT)], zv)

        def issue_idx(blk, s):
            base = blk * EB
            pltpu.async_copy(srci.at[pl.ds(base, EB)], si[s], smi[s])
            pltpu.async_copy(dsti.at[pl.ds(base, EB)], di[s], smi[s])
            pltpu.async_copy(
                w_hbm.at[pl.ds(base * PADH, EB * PADH)], wr[s], smi[s])

        def wait_idx(blk, s):
            base = blk * EB
            pltpu.make_async_copy(
                srci.at[pl.ds(base, EB)], si[s], smi[s]).wait()
            pltpu.make_async_copy(
                dsti.at[pl.ds(base, EB)], di[s], smi[s]).wait()
            pltpu.make_async_copy(
                w_hbm.at[pl.ds(base * PADH, EB * PADH)], wr[s], smi[s]).wait()

        for c in range(nchunk):

            @pl.when(c // chunks_per_core == cid)
            def _(c=c):
                hd = chunk_heads[c]
                for half in range(2):
                    lo = half * HNP
                    pltpu.sync_copy(zv, acc_sp.at[pl.ds(sid * HRT, HRT)])

                    @pl.when(sid == 0)
                    def _():
                        pltpu.sync_copy(zv.at[pl.ds(0, 8)],
                                        acc_sp.at[pl.ds(HNP, 8)])

                    plsc.subcore_barrier()
                    issue_idx(sid, 0)
                    issue_idx(NS + sid, 1)
                    wait_idx(sid, 0)
                    pltpu.async_copy(hc[c].at[si[0]], hr[0], smg[0])

                    def blkbody(i, carry):
                        s = lax.rem(i, 2)

                        def run_slot(s, c=c, hd=hd, lo=lo):
                            s1 = 1 - s
                            blk = i * NS + sid

                            @pl.when(i + 1 < _B2_ITERS)
                            def _():
                                wait_idx((i + 1) * NS + sid, s1)
                                pltpu.async_copy(
                                    hc[c].at[si[s1]], hr[s1], smg[s1])

                            pltpu.make_async_copy(
                                hc[c].at[si[s]], hr[s], smg[s]).wait()

                            def ebody(j, c2):
                                bidx = jnp.full((16,), j * PADH + hd, _I32)
                                sv = plsc.load_gather(wr[s], [bidx])
                                for f in range(CH // 16):
                                    sl = pl.ds(f * 16, 16)
                                    hr[s][j, sl] = hr[s][j, sl] * sv
                                return c2

                            lax.fori_loop(0, EB, ebody, 0)
                            for k in range(EB // 16):
                                sl = pl.ds(k * 16, 16)
                                t = di[s][sl] - lo
                                ok = (t >= 0) & (t < HNP)
                                dl[sl] = jnp.where(ok, t, HNP)

                            @pl.when(i + 2 < _B2_ITERS)
                            def _():
                                issue_idx((i + 2) * NS + sid, s)

                            pltpu.sync_copy(hr[s], acc_sp.at[dl], add=True)

                        @pl.when(s == 0)
                        def _():
                            run_slot(0)

                        @pl.when(s == 1)
                        def _():
                            run_slot(1)

                        return carry

                    lax.fori_loop(0, _B2_ITERS, blkbody, 0)
                    plsc.subcore_barrier()
                    pltpu.sync_copy(
                        acc_sp.at[pl.ds(sid * HRT, HRT)],
                        mout[c].at[pl.ds(lo + sid * HRT, HRT)])
                    plsc.subcore_barrier()

    def run(srci, dsti, w_hbm, zc, hchunks):
        return pl.kernel(
            body,
            out_type=[jax.ShapeDtypeStruct((NP, CH), _F32)] * nchunk,
            mesh=_MESH,
            compiler_params=pltpu.CompilerParams(needs_layout_passes=False),
            scratch_types=[
                pltpu.VMEM((EB,), _I32),
                pltpu.VMEM((EB,), _I32),
                pltpu.VMEM((EB,), _I32),
                pltpu.VMEM((EB,), _I32),
                pltpu.VMEM((EB,), _I32),
                pltpu.VMEM((EB * PADH,), _F32),
                pltpu.VMEM((EB * PADH,), _F32),
                pltpu.VMEM((EB, CH), _F32),
                pltpu.VMEM((EB, CH), _F32),
                pltpu.VMEM((HRT, CH), _F32),
                pltpu.SemaphoreType.DMA,
                pltpu.SemaphoreType.DMA,
                pltpu.SemaphoreType.DMA,
                pltpu.SemaphoreType.DMA,
                pltpu.VMEM_SHARED((ACC, CH), _F32),
            ],
        )(srci, dsti, w_hbm, zc, *hchunks)

    return run


_sc_b2_l1 = _make_sc_b2(8, (0, 0, 1, 1, 2, 2, 3, 3), 4)
_sc_b2_l2 = _make_sc_b2(2, (0, 0), 1)


@jax.jit
def kernel(x, edge_index, W1, a1_src, a1_dst, b1, W2, a2_src, a2_dst, b2):
    src = jnp.pad(edge_index[0].astype(_I32), (0, EPAD - E))
    dst = jnp.pad(edge_index[1].astype(_I32), (0, EPAD - E),
                  constant_values=NP - 1)

    eye = jnp.eye(HEADS, dtype=_F32)
    A1s = jnp.pad((eye[:, None, :] * a1_src.astype(_F32)[:, :, None])
                  .reshape(D1, HEADS), ((0, 0), (0, PADW - HEADS)))
    A1d = jnp.pad((eye[:, None, :] * a1_dst.astype(_F32)[:, :, None])
                  .reshape(D1, HEADS), ((0, 0), (0, PADW - HEADS)))
    A2s = jnp.pad(a2_src.astype(_F32).T, ((0, 0), (0, PADW - 1)))
    A2d = jnp.pad(a2_dst.astype(_F32).T, ((0, 0), (0, PADW - 1)))
    R1 = jnp.pad(jnp.repeat(eye, D1 // HEADS, axis=1),
                 ((0, DENS - HEADS), (0, 0)))
    R2 = jnp.pad(jnp.ones((1, DOUT), _F32), ((0, DENS - 1), (0, 0)))
    b1r = b1.astype(_F32).reshape(1, D1)
    b2r = b2.astype(_F32).reshape(1, DOUT)
    z8 = jnp.zeros((NP * DENS,), _F32)
    zc = jnp.zeros((RPT, CH), _F32)

    outs = _tc_a(x.astype(_F32), W1.astype(_F32), A1s, A1d)
    h1c, as1, ad1 = outs[:8], outs[8], outs[9]

    as1 = jnp.pad(as1, ((0, NP - N), (0, 0)))
    ad1 = jnp.pad(ad1, ((0, NP - N), (0, 0)))
    w1e, dp1 = _sc_b1(as1, ad1, src, dst, z8)
    dp1 = dp1.reshape(NW, NP, DENS)
    m1c = _sc_b2_l1(src, dst, w1e, zc, h1c)

    h2c0, h2c1, as2, ad2 = _tc_c(m1c, dp1, b1r, W2.astype(_F32),
                                 A2s, A2d, R1)

    as2 = jnp.pad(as2, ((0, NP - N), (0, 0)))
    ad2 = jnp.pad(ad2, ((0, NP - N), (0, 0)))
    w2e, dp2 = _sc_b1(as2, ad2, src, dst, z8)
    dp2 = dp2.reshape(NW, NP, DENS)
    m2c = _sc_b2_l2(src, dst, w2e, zc, (h2c0, h2c1))

    return _tc_e(m2c, dp2, b2r, R2)


# speedup vs baseline: 6.6806x; 1.1281x over previous
import jax
import jax.numpy as jnp
from jax import lax
from jax.experimental import pallas as pl
from jax.experimental.pallas import tpu as pltpu
from jax.experimental.pallas import tpu_sc as plsc

N = 10000
E = 160000
HEADS = 4
DIN = 256
D1 = 1024
DOUT = 256
PADH = 16
DENS = 8
CH = 128
PADW = 128
EB = 128
NBLK = E // EB
EPAD = 163840
NBLKP = EPAD // EB
NC = 2
NS = 16
NW = NC * NS
NP = 10240
RPT = NP // NS
HNP = NP // 2
HRT = HNP // NS
ACC = HNP + 8
RB = 400
GRID = N // RB
_F32 = jnp.float32
_I32 = jnp.int32


def _elu(x):
    return jnp.where(x > 0, x, jnp.exp(x) - 1.0)


def _tc_a_body(x_ref, w1_ref, a1s_ref, a1d_ref, *outs):
    h = jnp.dot(x_ref[...], w1_ref[...], preferred_element_type=_F32)
    for c in range(8):
        outs[c][...] = h[:, c * CH:(c + 1) * CH]
    outs[8][...] = jnp.dot(h, a1s_ref[...], preferred_element_type=_F32)
    outs[9][...] = jnp.dot(h, a1d_ref[...], preferred_element_type=_F32)


def _tc_c_body(m0, m1, m2, m3, m4, m5, m6, m7, dp_ref, b1_ref, w2_ref,
               a2s_ref, a2d_ref, r1_ref, h2c0, h2c1, as2, ad2):
    msg = jnp.concatenate([r[...] for r in (m0, m1, m2, m3, m4, m5, m6, m7)],
                          axis=1)
    d = jnp.sum(dp_ref[...], axis=0)
    r = 1.0 / (d + 1e-16)
    rep = jnp.dot(r, r1_ref[...], preferred_element_type=_F32)
    t = _elu(msg * rep + b1_ref[...])
    h2 = jnp.dot(t, w2_ref[...], preferred_element_type=_F32)
    h2c0[...] = h2[:, :CH]
    h2c1[...] = h2[:, CH:]
    as2[...] = jnp.dot(h2, a2s_ref[...], preferred_element_type=_F32)
    ad2[...] = jnp.dot(h2, a2d_ref[...], preferred_element_type=_F32)


def _tc_e_body(m0, m1, dp_ref, b2_ref, r2_ref, out_ref):
    msg = jnp.concatenate([m0[...], m1[...]], axis=1)
    d = jnp.sum(dp_ref[...], axis=0)
    r = 1.0 / (d + 1e-16)
    rep = jnp.dot(r, r2_ref[...], preferred_element_type=_F32)
    out_ref[...] = _elu(msg * rep + b2_ref[...])


def _rep_spec(shape):
    return pl.BlockSpec(shape, lambda i: tuple(0 for _ in shape))


def _tc_a(x, W1, A1s, A1d):
    return pl.pallas_call(
        _tc_a_body,
        grid=(GRID,),
        in_specs=[
            pl.BlockSpec((RB, DIN), lambda i: (i, 0)),
            _rep_spec((DIN, D1)),
            _rep_spec((D1, PADW)),
            _rep_spec((D1, PADW)),
        ],
        out_specs=[pl.BlockSpec((RB, CH), lambda i: (i, 0))] * 8
        + [pl.BlockSpec((RB, PADW), lambda i: (i, 0))] * 2,
        out_shape=[jax.ShapeDtypeStruct((N, CH), _F32)] * 8
        + [jax.ShapeDtypeStruct((N, PADW), _F32)] * 2,
    )(x, W1, A1s, A1d)


def _tc_c(m1c, dp1, b1r, W2, A2s, A2d, R1):
    return pl.pallas_call(
        _tc_c_body,
        grid=(GRID,),
        in_specs=[pl.BlockSpec((RB, CH), lambda i: (i, 0))] * 8 + [
            pl.BlockSpec((NW, RB, DENS), lambda i: (0, i, 0)),
            _rep_spec((1, D1)),
            _rep_spec((D1, DOUT)),
            _rep_spec((DOUT, PADW)),
            _rep_spec((DOUT, PADW)),
            _rep_spec((DENS, D1)),
        ],
        out_specs=[pl.BlockSpec((RB, CH), lambda i: (i, 0))] * 2
        + [pl.BlockSpec((RB, PADW), lambda i: (i, 0))] * 2,
        out_shape=[jax.ShapeDtypeStruct((N, CH), _F32)] * 2
        + [jax.ShapeDtypeStruct((N, PADW), _F32)] * 2,
    )(*m1c, dp1, b1r, W2, A2s, A2d, R1)


def _tc_e(m2c, dp2, b2r, R2):
    return pl.pallas_call(
        _tc_e_body,
        grid=(GRID,),
        in_specs=[pl.BlockSpec((RB, CH), lambda i: (i, 0))] * 2 + [
            pl.BlockSpec((NW, RB, DENS), lambda i: (0, i, 0)),
            _rep_spec((1, DOUT)),
            _rep_spec((DENS, DOUT)),
        ],
        out_specs=pl.BlockSpec((RB, DOUT), lambda i: (i, 0)),
        out_shape=jax.ShapeDtypeStruct((N, DOUT), _F32),
    )(*m2c, dp2, b2r, R2)


_MESH = plsc.VectorSubcoreMesh(core_axis_name="c", subcore_axis_name="s")
_B1_ITERS = -(-NBLK // NW)
_B2_ITERS = NBLKP // NS


def _sc_b1_body(asrc, adst, srci, dsti, z8, w_hbm, dpart,
                si, di, ar, dr, ws, den_v):
    cid = lax.axis_index("c")
    sid = lax.axis_index("s")
    wid = sid * NC + cid
    pltpu.sync_copy(z8, den_v)

    def blkbody(i, carry):
        blk = i * NW + wid

        @pl.when(blk < NBLK)
        def _():
            base = blk * EB
            pltpu.sync_copy(srci.at[pl.ds(base, EB)], si)
            pltpu.sync_copy(dsti.at[pl.ds(base, EB)], di)
            pltpu.sync_copy(asrc.at[si], ar)
            pltpu.sync_copy(adst.at[di], dr)
            for g in range(EB // 16):
                rows = lax.iota(_I32, 16) + (g * 16)
                di16 = di[pl.ds(g * 16, 16)]
                for h in range(HEADS):
                    colh = jnp.full((16,), h, _I32)
                    av = plsc.load_gather(ar, [rows, colh])
                    dv = plsc.load_gather(dr, [rows, colh])
                    v = av + dv
                    v = jnp.where(v > 0, v, 0.2 * v)
                    w = jnp.exp(v)
                    plsc.addupdate_scatter(den_v, [di16 * DENS + h], w)
                    plsc.store_scatter(ws, [rows * PADH + h], w)
            pltpu.sync_copy(ws, w_hbm.at[pl.ds(base * PADH, EB * PADH)])
        return carry

    lax.fori_loop(0, _B1_ITERS, blkbody, 0)
    pltpu.sync_copy(den_v, dpart.at[wid])


def _sc_b1(asrc, adst, srci, dsti, z8):
    return pl.kernel(
        _sc_b1_body,
        out_type=[
            jax.ShapeDtypeStruct((EPAD * PADH,), _F32),
            jax.ShapeDtypeStruct((NW, NP * DENS), _F32),
        ],
        mesh=_MESH,
        compiler_params=pltpu.CompilerParams(needs_layout_passes=False),
        scratch_types=[
            pltpu.VMEM((EB,), _I32),
            pltpu.VMEM((EB,), _I32),
            pltpu.VMEM((EB, PADW), _F32),
            pltpu.VMEM((EB, PADW), _F32),
            pltpu.VMEM((EB * PADH,), _F32),
            pltpu.VMEM((NP * DENS,), _F32),
        ],
    )(asrc, adst, srci, dsti, z8)


def _make_sc_b2(nchunk, chunk_heads, chunks_per_core):
    def body(srci, dsti, w_hbm, zc, *rest):
        hc = rest[:nchunk]
        mout = rest[nchunk:2 * nchunk]
        (si0, si1, di0, di1, dl0, dl1, wr0, wr1, hr0, hr1, zv,
         smi0, smi1, smg0, smg1, sms0, sms1, acc_sp) = rest[2 * nchunk:]
        si = (si0, si1)
        di = (di0, di1)
        dl = (dl0, dl1)
        wr = (wr0, wr1)
        hr = (hr0, hr1)
        smi = (smi0, smi1)
        smg = (smg0, smg1)
        sms = (sms0, sms1)
        cid = lax.axis_index("c")
        sid = lax.axis_index("s")
        pltpu.sync_copy(zc.at[pl.ds(0, HRT)], zv)

        def issue_idx(blk, s):
            base = blk * EB
            pltpu.async_copy(srci.at[pl.ds(base, EB)], si[s], smi[s])
            pltpu.async_copy(dsti.at[pl.ds(base, EB)], di[s], smi[s])
            pltpu.async_copy(
                w_hbm.at[pl.ds(base * PADH, EB * PADH)], wr[s], smi[s])

        def wait_idx(blk, s):
            base = blk * EB
            pltpu.make_async_copy(
                srci.at[pl.ds(base, EB)], si[s], smi[s]).wait()
            pltpu.make_async_copy(
                dsti.at[pl.ds(base, EB)], di[s], smi[s]).wait()
            pltpu.make_async_copy(
                w_hbm.at[pl.ds(base * PADH, EB * PADH)], wr[s], smi[s]).wait()

        for c in range(nchunk):

            @pl.when(c // chunks_per_core == cid)
            def _(c=c):
                hd = chunk_heads[c]
                for half in range(2):
                    lo = half * HNP
                    pltpu.sync_copy(zv, acc_sp.at[pl.ds(sid * HRT, HRT)])

                    @pl.when(sid == 0)
                    def _():
                        pltpu.sync_copy(zv.at[pl.ds(0, 8)],
                                        acc_sp.at[pl.ds(HNP, 8)])

                    plsc.subcore_barrier()
                    issue_idx(sid, 0)
                    issue_idx(NS + sid, 1)
                    wait_idx(sid, 0)
                    pltpu.async_copy(hc[c].at[si[0]], hr[0], smg[0])

                    def blkbody(i, carry):
                        s = lax.rem(i, 2)

                        def run_slot(s, c=c, hd=hd, lo=lo):
                            s1 = 1 - s

                            @pl.when(i + 1 < _B2_ITERS)
                            def _():
                                wait_idx((i + 1) * NS + sid, s1)

                                @pl.when(i >= 1)
                                def _():
                                    pltpu.make_async_copy(
                                        hr[s1], acc_sp.at[dl[s1]],
                                        sms[s1]).wait()

                                pltpu.async_copy(
                                    hc[c].at[si[s1]], hr[s1], smg[s1])

                            pltpu.make_async_copy(
                                hc[c].at[si[s]], hr[s], smg[s]).wait()

                            def ebody(j, c2):
                                bidx = jnp.full((16,), j * PADH + hd, _I32)
                                sv = plsc.load_gather(wr[s], [bidx])
                                for f in range(CH // 16):
                                    sl = pl.ds(f * 16, 16)
                                    hr[s][j, sl] = hr[s][j, sl] * sv
                                return c2

                            lax.fori_loop(0, EB, ebody, 0)
                            for k in range(EB // 16):
                                sl = pl.ds(k * 16, 16)
                                t = di[s][sl] - lo
                                ok = (t >= 0) & (t < HNP)
                                dl[s][sl] = jnp.where(ok, t, HNP)

                            @pl.when(i + 2 < _B2_ITERS)
                            def _():
                                issue_idx((i + 2) * NS + sid, s)

                            pltpu.async_copy(
                                hr[s], acc_sp.at[dl[s]], sms[s], add=True)

                        @pl.when(s == 0)
                        def _():
                            run_slot(0)

                        @pl.when(s == 1)
                        def _():
                            run_slot(1)

                        return carry

                    lax.fori_loop(0, _B2_ITERS, blkbody, 0)
                    sL = (_B2_ITERS - 1) % 2
                    pltpu.make_async_copy(
                        hr[sL], acc_sp.at[dl[sL]], sms[sL]).wait()
                    pltpu.make_async_copy(
                        hr[1 - sL], acc_sp.at[dl[1 - sL]],
                        sms[1 - sL]).wait()
                    plsc.subcore_barrier()
                    pltpu.sync_copy(
                        acc_sp.at[pl.ds(sid * HRT, HRT)],
                        mout[c].at[pl.ds(lo + sid * HRT, HRT)])
                    plsc.subcore_barrier()

    def run(srci, dsti, w_hbm, zc, hchunks):
        return pl.kernel(
            body,
            out_type=[jax.ShapeDtypeStruct((NP, CH), _F32)] * nchunk,
            mesh=_MESH,
            compiler_params=pltpu.CompilerParams(needs_layout_passes=False),
            scratch_types=[
                pltpu.VMEM((EB,), _I32),
                pltpu.VMEM((EB,), _I32),
                pltpu.VMEM((EB,), _I32),
                pltpu.VMEM((EB,), _I32),
                pltpu.VMEM((EB,), _I32),
                pltpu.VMEM((EB,), _I32),
                pltpu.VMEM((EB * PADH,), _F32),
                pltpu.VMEM((EB * PADH,), _F32),
                pltpu.VMEM((EB, CH), _F32),
                pltpu.VMEM((EB, CH), _F32),
                pltpu.VMEM((HRT, CH), _F32),
                pltpu.SemaphoreType.DMA,
                pltpu.SemaphoreType.DMA,
                pltpu.SemaphoreType.DMA,
                pltpu.SemaphoreType.DMA,
                pltpu.SemaphoreType.DMA,
                pltpu.SemaphoreType.DMA,
                pltpu.VMEM_SHARED((ACC, CH), _F32),
            ],
        )(srci, dsti, w_hbm, zc, *hchunks)

    return run


_sc_b2_l1 = _make_sc_b2(8, (0, 0, 1, 1, 2, 2, 3, 3), 4)
_sc_b2_l2 = _make_sc_b2(2, (0, 0), 1)


@jax.jit
def kernel(x, edge_index, W1, a1_src, a1_dst, b1, W2, a2_src, a2_dst, b2):
    src = jnp.pad(edge_index[0].astype(_I32), (0, EPAD - E))
    dst = jnp.pad(edge_index[1].astype(_I32), (0, EPAD - E),
                  constant_values=NP - 1)

    eye = jnp.eye(HEADS, dtype=_F32)
    A1s = jnp.pad((eye[:, None, :] * a1_src.astype(_F32)[:, :, None])
                  .reshape(D1, HEADS), ((0, 0), (0, PADW - HEADS)))
    A1d = jnp.pad((eye[:, None, :] * a1_dst.astype(_F32)[:, :, None])
                  .reshape(D1, HEADS), ((0, 0), (0, PADW - HEADS)))
    A2s = jnp.pad(a2_src.astype(_F32).T, ((0, 0), (0, PADW - 1)))
    A2d = jnp.pad(a2_dst.astype(_F32).T, ((0, 0), (0, PADW - 1)))
    R1 = jnp.pad(jnp.repeat(eye, D1 // HEADS, axis=1),
                 ((0, DENS - HEADS), (0, 0)))
    R2 = jnp.pad(jnp.ones((1, DOUT), _F32), ((0, DENS - 1), (0, 0)))
    b1r = b1.astype(_F32).reshape(1, D1)
    b2r = b2.astype(_F32).reshape(1, DOUT)
    z8 = jnp.zeros((NP * DENS,), _F32)
    zc = jnp.zeros((RPT, CH), _F32)

    outs = _tc_a(x.astype(_F32), W1.astype(_F32), A1s, A1d)
    h1c, as1, ad1 = outs[:8], outs[8], outs[9]

    w1e, dp1 = _sc_b1(as1, ad1, src, dst, z8)
    dp1 = dp1.reshape(NW, NP, DENS)
    m1c = _sc_b2_l1(src, dst, w1e, zc, h1c)

    h2c0, h2c1, as2, ad2 = _tc_c(m1c, dp1, b1r, W2.astype(_F32),
                                 A2s, A2d, R1)

    w2e, dp2 = _sc_b1(as2, ad2, src, dst, z8)
    dp2 = dp2.reshape(NW, NP, DENS)
    m2c = _sc_b2_l2(src, dst, w2e, zc, (h2c0, h2c1))

    return _tc_e(m2c, dp2, b2r, R2)


# speedup vs baseline: 6.8514x; 1.0256x over previous
import jax
import jax.numpy as jnp
from jax import lax
from jax.experimental import pallas as pl
from jax.experimental.pallas import tpu as pltpu
from jax.experimental.pallas import tpu_sc as plsc

N = 10000
E = 160000
HEADS = 4
DIN = 256
D1 = 1024
DOUT = 256
PADH = 16
DENS = 8
CH = 128
PADW = 128
EB = 128
NBLK = E // EB
EPAD = 163840
NBLKP = EPAD // EB
NC = 2
NS = 16
NW = NC * NS
NP = 10240
RPT = NP // NS
HNP = NP // 2
HRT = HNP // NS
ACC = HNP + 8
RB = 400
GRID = N // RB
_F32 = jnp.float32
_I32 = jnp.int32


def _elu(x):
    return jnp.where(x > 0, x, jnp.exp(x) - 1.0)


def _tc_a_body(x_ref, w1_ref, a1s_ref, a1d_ref, *outs):
    h = jnp.dot(x_ref[...], w1_ref[...], preferred_element_type=_F32)
    for c in range(8):
        outs[c][...] = h[:, c * CH:(c + 1) * CH]
    outs[8][...] = jnp.dot(h, a1s_ref[...], preferred_element_type=_F32)
    outs[9][...] = jnp.dot(h, a1d_ref[...], preferred_element_type=_F32)


def _tc_c_body(m0, m1, m2, m3, m4, m5, m6, m7, dp_ref, b1_ref, w2_ref,
               a2s_ref, a2d_ref, r1_ref, h2c0, h2c1, as2, ad2):
    msg = jnp.concatenate([r[...] for r in (m0, m1, m2, m3, m4, m5, m6, m7)],
                          axis=1)
    d = jnp.sum(dp_ref[...], axis=0)
    r = 1.0 / (d + 1e-16)
    rep = jnp.dot(r, r1_ref[...], preferred_element_type=_F32)
    t = _elu(msg * rep + b1_ref[...])
    h2 = jnp.dot(t, w2_ref[...], preferred_element_type=_F32)
    h2c0[...] = h2[:, :CH]
    h2c1[...] = h2[:, CH:]
    as2[...] = jnp.dot(h2, a2s_ref[...], preferred_element_type=_F32)
    ad2[...] = jnp.dot(h2, a2d_ref[...], preferred_element_type=_F32)


def _tc_e_body(m0, m1, dp_ref, b2_ref, r2_ref, out_ref):
    msg = jnp.concatenate([m0[...], m1[...]], axis=1)
    d = jnp.sum(dp_ref[...], axis=0)
    r = 1.0 / (d + 1e-16)
    rep = jnp.dot(r, r2_ref[...], preferred_element_type=_F32)
    out_ref[...] = _elu(msg * rep + b2_ref[...])


def _rep_spec(shape):
    return pl.BlockSpec(shape, lambda i: tuple(0 for _ in shape))


def _tc_a(x, W1, A1s, A1d):
    return pl.pallas_call(
        _tc_a_body,
        grid=(GRID,),
        in_specs=[
            pl.BlockSpec((RB, DIN), lambda i: (i, 0)),
            _rep_spec((DIN, D1)),
            _rep_spec((D1, PADW)),
            _rep_spec((D1, PADW)),
        ],
        out_specs=[pl.BlockSpec((RB, CH), lambda i: (i, 0))] * 8
        + [pl.BlockSpec((RB, PADW), lambda i: (i, 0))] * 2,
        out_shape=[jax.ShapeDtypeStruct((N, CH), _F32)] * 8
        + [jax.ShapeDtypeStruct((N, PADW), _F32)] * 2,
    )(x, W1, A1s, A1d)


def _tc_c(m1c, dp1, b1r, W2, A2s, A2d, R1):
    return pl.pallas_call(
        _tc_c_body,
        grid=(GRID,),
        in_specs=[pl.BlockSpec((RB, CH), lambda i: (i, 0))] * 8 + [
            pl.BlockSpec((NW, RB, DENS), lambda i: (0, i, 0)),
            _rep_spec((1, D1)),
            _rep_spec((D1, DOUT)),
            _rep_spec((DOUT, PADW)),
            _rep_spec((DOUT, PADW)),
            _rep_spec((DENS, D1)),
        ],
        out_specs=[pl.BlockSpec((RB, CH), lambda i: (i, 0))] * 2
        + [pl.BlockSpec((RB, PADW), lambda i: (i, 0))] * 2,
        out_shape=[jax.ShapeDtypeStruct((N, CH), _F32)] * 2
        + [jax.ShapeDtypeStruct((N, PADW), _F32)] * 2,
    )(*m1c, dp1, b1r, W2, A2s, A2d, R1)


def _tc_e(m2c, dp2, b2r, R2):
    return pl.pallas_call(
        _tc_e_body,
        grid=(GRID,),
        in_specs=[pl.BlockSpec((RB, CH), lambda i: (i, 0))] * 2 + [
            pl.BlockSpec((NW, RB, DENS), lambda i: (0, i, 0)),
            _rep_spec((1, DOUT)),
            _rep_spec((DENS, DOUT)),
        ],
        out_specs=pl.BlockSpec((RB, DOUT), lambda i: (i, 0)),
        out_shape=jax.ShapeDtypeStruct((N, DOUT), _F32),
    )(*m2c, dp2, b2r, R2)


_MESH = plsc.VectorSubcoreMesh(core_axis_name="c", subcore_axis_name="s")
_B1_ITERS = -(-NBLK // NW)
_B2_ITERS = NBLKP // NS


def _sc_b1_body(asrc, adst, srci, dsti, z8, w_hbm, dpart,
                si, di, ar, dr, ws, den_v, smb):
    cid = lax.axis_index("c")
    sid = lax.axis_index("s")
    wid = sid * NC + cid
    pltpu.sync_copy(z8, den_v)

    def blkbody(i, carry):
        blk = i * NW + wid

        @pl.when(blk < NBLK)
        def _():
            base = blk * EB
            pltpu.async_copy(srci.at[pl.ds(base, EB)], si, smb)
            pltpu.async_copy(dsti.at[pl.ds(base, EB)], di, smb)
            pltpu.make_async_copy(srci.at[pl.ds(base, EB)], si, smb).wait()
            pltpu.make_async_copy(dsti.at[pl.ds(base, EB)], di, smb).wait()
            pltpu.async_copy(asrc.at[si], ar, smb)
            pltpu.async_copy(adst.at[di], dr, smb)
            pltpu.make_async_copy(asrc.at[si], ar, smb).wait()
            pltpu.make_async_copy(adst.at[di], dr, smb).wait()
            for g in range(EB // 16):
                rows = lax.iota(_I32, 16) + (g * 16)
                di16 = di[pl.ds(g * 16, 16)]
                for h in range(HEADS):
                    colh = jnp.full((16,), h, _I32)
                    av = plsc.load_gather(ar, [rows, colh])
                    dv = plsc.load_gather(dr, [rows, colh])
                    v = av + dv
                    v = jnp.where(v > 0, v, 0.2 * v)
                    w = jnp.exp(v)
                    plsc.addupdate_scatter(den_v, [di16 * DENS + h], w)
                    plsc.store_scatter(ws, [rows * PADH + h], w)
            pltpu.sync_copy(ws, w_hbm.at[pl.ds(base * PADH, EB * PADH)])
        return carry

    lax.fori_loop(0, _B1_ITERS, blkbody, 0)
    pltpu.sync_copy(den_v, dpart.at[wid])


def _sc_b1(asrc, adst, srci, dsti, z8):
    return pl.kernel(
        _sc_b1_body,
        out_type=[
            jax.ShapeDtypeStruct((EPAD * PADH,), _F32),
            jax.ShapeDtypeStruct((NW, NP * DENS), _F32),
        ],
        mesh=_MESH,
        compiler_params=pltpu.CompilerParams(needs_layout_passes=False),
        scratch_types=[
            pltpu.VMEM((EB,), _I32),
            pltpu.VMEM((EB,), _I32),
            pltpu.VMEM((EB, PADW), _F32),
            pltpu.VMEM((EB, PADW), _F32),
            pltpu.VMEM((EB * PADH,), _F32),
            pltpu.VMEM((NP * DENS,), _F32),
            pltpu.SemaphoreType.DMA,
        ],
    )(asrc, adst, srci, dsti, z8)


def _make_sc_b2(nchunk, chunk_heads, chunks_per_core):
    def body(srci, dsti, w_hbm, zc, *rest):
        hc = rest[:nchunk]
        mout = rest[nchunk:2 * nchunk]
        (si0, si1, di0, di1, dl0, dl1, wr0, wr1, hr0, hr1, zv,
         smi0, smi1, smg0, smg1, sms0, sms1, acc_sp) = rest[2 * nchunk:]
        si = (si0, si1)
        di = (di0, di1)
        dl = (dl0, dl1)
        wr = (wr0, wr1)
        hr = (hr0, hr1)
        smi = (smi0, smi1)
        smg = (smg0, smg1)
        sms = (sms0, sms1)
        cid = lax.axis_index("c")
        sid = lax.axis_index("s")
        pltpu.sync_copy(zc.at[pl.ds(0, HRT)], zv)

        def issue_idx(blk, s):
            base = blk * EB
            pltpu.async_copy(srci.at[pl.ds(base, EB)], si[s], smi[s])
            pltpu.async_copy(dsti.at[pl.ds(base, EB)], di[s], smi[s])
            pltpu.async_copy(
                w_hbm.at[pl.ds(base * PADH, EB * PADH)], wr[s], smi[s])

        def wait_idx(blk, s):
            base = blk * EB
            pltpu.make_async_copy(
                srci.at[pl.ds(base, EB)], si[s], smi[s]).wait()
            pltpu.make_async_copy(
                dsti.at[pl.ds(base, EB)], di[s], smi[s]).wait()
            pltpu.make_async_copy(
                w_hbm.at[pl.ds(base * PADH, EB * PADH)], wr[s], smi[s]).wait()

        for c in range(nchunk):

            @pl.when(c // chunks_per_core == cid)
            def _(c=c):
                hd = chunk_heads[c]
                for half in range(2):
                    lo = half * HNP
                    pltpu.sync_copy(zv, acc_sp.at[pl.ds(sid * HRT, HRT)])

                    @pl.when(sid == 0)
                    def _():
                        pltpu.sync_copy(zv.at[pl.ds(0, 8)],
                                        acc_sp.at[pl.ds(HNP, 8)])

                    plsc.subcore_barrier()
                    issue_idx(sid, 0)
                    issue_idx(NS + sid, 1)
                    wait_idx(sid, 0)
                    pltpu.async_copy(hc[c].at[si[0]], hr[0], smg[0])

                    def blkbody(i, carry):
                        s = lax.rem(i, 2)

                        def run_slot(s, c=c, hd=hd, lo=lo):
                            s1 = 1 - s

                            @pl.when(i + 1 < _B2_ITERS)
                            def _():
                                wait_idx((i + 1) * NS + sid, s1)

                                @pl.when(i >= 1)
                                def _():
                                    pltpu.make_async_copy(
                                        hr[s1], acc_sp.at[dl[s1]],
                                        sms[s1]).wait()

                                pltpu.async_copy(
                                    hc[c].at[si[s1]], hr[s1], smg[s1])

                            pltpu.make_async_copy(
                                hc[c].at[si[s]], hr[s], smg[s]).wait()

                            def ebody(j, c2):
                                bidx = jnp.full((16,), j * PADH + hd, _I32)
                                sv = plsc.load_gather(wr[s], [bidx])
                                for f in range(CH // 16):
                                    sl = pl.ds(f * 16, 16)
                                    hr[s][j, sl] = hr[s][j, sl] * sv
                                return c2

                            lax.fori_loop(0, EB, ebody, 0, unroll=4)
                            for k in range(EB // 16):
                                sl = pl.ds(k * 16, 16)
                                t = di[s][sl] - lo
                                ok = (t >= 0) & (t < HNP)
                                dl[s][sl] = jnp.where(ok, t, HNP)

                            @pl.when(i + 2 < _B2_ITERS)
                            def _():
                                issue_idx((i + 2) * NS + sid, s)

                            pltpu.async_copy(
                                hr[s], acc_sp.at[dl[s]], sms[s], add=True)

                        @pl.when(s == 0)
                        def _():
                            run_slot(0)

                        @pl.when(s == 1)
                        def _():
                            run_slot(1)

                        return carry

                    lax.fori_loop(0, _B2_ITERS, blkbody, 0)
                    sL = (_B2_ITERS - 1) % 2
                    pltpu.make_async_copy(
                        hr[sL], acc_sp.at[dl[sL]], sms[sL]).wait()
                    pltpu.make_async_copy(
                        hr[1 - sL], acc_sp.at[dl[1 - sL]],
                        sms[1 - sL]).wait()
                    plsc.subcore_barrier()
                    pltpu.sync_copy(
                        acc_sp.at[pl.ds(sid * HRT, HRT)],
                        mout[c].at[pl.ds(lo + sid * HRT, HRT)])
                    plsc.subcore_barrier()

    def run(srci, dsti, w_hbm, zc, hchunks):
        return pl.kernel(
            body,
            out_type=[jax.ShapeDtypeStruct((NP, CH), _F32)] * nchunk,
            mesh=_MESH,
            compiler_params=pltpu.CompilerParams(needs_layout_passes=False),
            scratch_types=[
                pltpu.VMEM((EB,), _I32),
                pltpu.VMEM((EB,), _I32),
                pltpu.VMEM((EB,), _I32),
                pltpu.VMEM((EB,), _I32),
                pltpu.VMEM((EB,), _I32),
                pltpu.VMEM((EB,), _I32),
                pltpu.VMEM((EB * PADH,), _F32),
                pltpu.VMEM((EB * PADH,), _F32),
                pltpu.VMEM((EB, CH), _F32),
                pltpu.VMEM((EB, CH), _F32),
                pltpu.VMEM((HRT, CH), _F32),
                pltpu.SemaphoreType.DMA,
                pltpu.SemaphoreType.DMA,
                pltpu.SemaphoreType.DMA,
                pltpu.SemaphoreType.DMA,
                pltpu.SemaphoreType.DMA,
                pltpu.SemaphoreType.DMA,
                pltpu.VMEM_SHARED((ACC, CH), _F32),
            ],
        )(srci, dsti, w_hbm, zc, *hchunks)

    return run


_sc_b2_l1 = _make_sc_b2(8, (0, 0, 1, 1, 2, 2, 3, 3), 4)
_sc_b2_l2 = _make_sc_b2(2, (0, 0), 1)


@jax.jit
def kernel(x, edge_index, W1, a1_src, a1_dst, b1, W2, a2_src, a2_dst, b2):
    src = jnp.pad(edge_index[0].astype(_I32), (0, EPAD - E))
    dst = jnp.pad(edge_index[1].astype(_I32), (0, EPAD - E),
                  constant_values=NP - 1)

    eye = jnp.eye(HEADS, dtype=_F32)
    A1s = jnp.pad((eye[:, None, :] * a1_src.astype(_F32)[:, :, None])
                  .reshape(D1, HEADS), ((0, 0), (0, PADW - HEADS)))
    A1d = jnp.pad((eye[:, None, :] * a1_dst.astype(_F32)[:, :, None])
                  .reshape(D1, HEADS), ((0, 0), (0, PADW - HEADS)))
    A2s = jnp.pad(a2_src.astype(_F32).T, ((0, 0), (0, PADW - 1)))
    A2d = jnp.pad(a2_dst.astype(_F32).T, ((0, 0), (0, PADW - 1)))
    R1 = jnp.pad(jnp.repeat(eye, D1 // HEADS, axis=1),
                 ((0, DENS - HEADS), (0, 0)))
    R2 = jnp.pad(jnp.ones((1, DOUT), _F32), ((0, DENS - 1), (0, 0)))
    b1r = b1.astype(_F32).reshape(1, D1)
    b2r = b2.astype(_F32).reshape(1, DOUT)
    z8 = jnp.zeros((NP * DENS,), _F32)
    zc = jnp.zeros((RPT, CH), _F32)

    outs = _tc_a(x.astype(_F32), W1.astype(_F32), A1s, A1d)
    h1c, as1, ad1 = outs[:8], outs[8], outs[9]

    w1e, dp1 = _sc_b1(as1, ad1, src, dst, z8)
    dp1 = dp1.reshape(NW, NP, DENS)
    m1c = _sc_b2_l1(src, dst, w1e, zc, h1c)

    h2c0, h2c1, as2, ad2 = _tc_c(m1c, dp1, b1r, W2.astype(_F32),
                                 A2s, A2d, R1)

    w2e, dp2 = _sc_b1(as2, ad2, src, dst, z8)
    dp2 = dp2.reshape(NW, NP, DENS)
    m2c = _sc_b2_l2(src, dst, w2e, zc, (h2c0, h2c1))

    return _tc_e(m2c, dp2, b2r, R2)
